# baseline (speedup 1.0000x reference)
import functools

import jax
import jax.numpy as jnp
from jax import lax
from jax.experimental import pallas as pl
from jax.experimental.pallas import tpu as pltpu
from jax.experimental.pallas import tpu_sc as plsc

N = 10000
D = 2
I = 32
T = 128
P = 100000
ND = N * D
NDP = 20480
NB = 5120
NC = 2
NS = 16
NW = NC * NS
TPW = T // NW
CP = 4096
NCH = 25


def _tc_body(times_ref, vp_ref, x0p_ref, u_ref):
    t = times_ref[:]
    j = lax.broadcasted_iota(jnp.int32, (T, I), 1).astype(jnp.float32)
    s = jnp.float32(1.0 / I)
    w = jnp.clip(t - j * s, 0.0, s)
    u = lax.dot_general(w, vp_ref[:], (((1,), (0,)), ((), ())),
                        precision=lax.Precision.HIGHEST,
                        preferred_element_type=jnp.float32)
    u_ref[:] = u + x0p_ref[:]


def _sc_pair_kernel(u, pa, pb):
    mesh = plsc.VectorSubcoreMesh(core_axis_name="c", subcore_axis_name="s")

    @functools.partial(
        pl.kernel,
        mesh=mesh,
        compiler_params=pltpu.CompilerParams(
            use_tc_tiling_on_sc=False, needs_layout_passes=False),
        out_type=jax.ShapeDtypeStruct((T, P), jnp.float32),
        scratch_types=[
            pltpu.VMEM((TPW * NDP,), jnp.float32),
            pltpu.VMEM((CP,), jnp.int32),
            pltpu.VMEM((CP,), jnp.int32),
            pltpu.VMEM((TPW, CP), jnp.float32),
        ],
    )
    def body(u_hbm, pa_hbm, pb_hbm, out_hbm, tab, ia, ib, obuf):
        wid = lax.axis_index("s") * NC + lax.axis_index("c")
        t0 = wid * TPW
        for k in range(TPW):
            pltpu.sync_copy(u_hbm.at[t0 + k], tab.at[pl.ds(k * NDP, NDP)])

        def chunk_step(c, carry):
            base = jnp.minimum(c * CP, P - CP)
            pltpu.sync_copy(pa_hbm.at[pl.ds(base, CP)], ia)
            pltpu.sync_copy(pb_hbm.at[pl.ds(base, CP)], ib)

            bsq = [tab[pl.ds(k * NDP + ND, 16)] for k in range(TPW)]

            def group_step(g, carry2):
                sl = pl.ds(g * 16, 16)
                a2 = ia[sl] * 2
                b2 = ib[sl] * 2
                for k in range(TPW):
                    off = k * NDP
                    ua0 = plsc.load_gather(tab, [a2 + off])
                    ua1 = plsc.load_gather(tab, [a2 + (off + 1)])
                    ub0 = plsc.load_gather(tab, [b2 + off])
                    ub1 = plsc.load_gather(tab, [b2 + (off + 1)])
                    d0 = ua0 - ub0
                    d1 = ua1 - ub1
                    obuf[k, sl] = jnp.exp(bsq[k] - (d0 * d0 + d1 * d1))
                return carry2

            lax.fori_loop(0, CP // 16, group_step, 0)
            for k in range(TPW):
                pltpu.sync_copy(obuf.at[k], out_hbm.at[t0 + k, pl.ds(base, CP)])
            return carry

        lax.fori_loop(0, NCH, chunk_step, 0)

    return body(u, pa, pb)


def kernel(x0, v, beta, times_list, node_pairs):
    vp = jnp.pad(v.reshape(I, ND), ((0, 0), (0, NDP - ND)))
    bsq = beta[0] * beta[0]
    x0p = jnp.concatenate([
        x0.reshape(1, ND),
        jnp.full((1, 16), bsq, jnp.float32),
        jnp.zeros((1, NDP - ND - 16), jnp.float32),
    ], axis=1)

    u = pl.pallas_call(
        _tc_body,
        grid=(NDP // NB,),
        in_specs=[
            pl.BlockSpec((T, 1), lambda i: (0, 0)),
            pl.BlockSpec((I, NB), lambda i: (0, i)),
            pl.BlockSpec((1, NB), lambda i: (0, i)),
        ],
        out_specs=pl.BlockSpec((T, NB), lambda i: (0, i)),
        out_shape=jax.ShapeDtypeStruct((T, NDP), jnp.float32),
    )(times_list.reshape(T, 1), vp, x0p)

    return _sc_pair_kernel(u, node_pairs[0], node_pairs[1])


# speedup vs baseline: 1.7075x; 1.7075x over previous
import functools

import jax
import jax.numpy as jnp
from jax import lax
from jax.experimental import pallas as pl
from jax.experimental.pallas import tpu as pltpu
from jax.experimental.pallas import tpu_sc as plsc

N = 10000
D = 2
I = 32
T = 128
P = 100000
ND = N * D
NDP = 20480
NB = 5120
NC = 2
NS = 16
NW = NC * NS
TPW = T // NW
CP = 4096
NCH = 25


def _tc_body(times_ref, vp_ref, x0p_ref, u_ref):
    t = times_ref[:]
    j = lax.broadcasted_iota(jnp.int32, (T, I), 1).astype(jnp.float32)
    s = jnp.float32(1.0 / I)
    w = jnp.clip(t - j * s, 0.0, s)
    u = lax.dot_general(w, vp_ref[:], (((1,), (0,)), ((), ())),
                        precision=lax.Precision.HIGHEST,
                        preferred_element_type=jnp.float32)
    u_ref[:] = u + x0p_ref[:]


def _sc_pair_kernel(u, pa, pb):
    mesh = plsc.VectorSubcoreMesh(core_axis_name="c", subcore_axis_name="s")

    @functools.partial(
        pl.kernel,
        mesh=mesh,
        compiler_params=pltpu.CompilerParams(
            use_tc_tiling_on_sc=False, needs_layout_passes=False),
        out_type=jax.ShapeDtypeStruct((T, P), jnp.float32),
        scratch_types=[
            pltpu.VMEM((TPW * NDP,), jnp.float32),
            pltpu.VMEM((CP,), jnp.int32),
            pltpu.VMEM((CP,), jnp.int32),
            pltpu.VMEM((TPW, CP), jnp.float32),
        ],
    )
    def body(u_hbm, pa_hbm, pb_hbm, out_hbm, tab, ia, ib, obuf):
        wid = lax.axis_index("s") * NC + lax.axis_index("c")
        t0 = wid * TPW
        for k in range(TPW):
            pltpu.sync_copy(u_hbm.at[t0 + k], tab.at[pl.ds(k * NDP, NDP)])

        bsq = [tab[pl.ds(k * NDP + ND, 16)] for k in range(TPW)]

        def chunk_step(c, carry):
            base = jnp.minimum(c * CP, P - CP)
            pltpu.sync_copy(pa_hbm.at[pl.ds(base, CP)], ia)
            pltpu.sync_copy(pb_hbm.at[pl.ds(base, CP)], ib)

            @plsc.parallel_loop(0, CP // 16, unroll=8)
            def group_step(g):
                sl = pl.ds(g * 16, 16)
                a2 = ia[sl] * 2
                b2 = ib[sl] * 2
                for k in range(TPW):
                    off = k * NDP
                    ua0 = plsc.load_gather(tab, [a2 + off])
                    ua1 = plsc.load_gather(tab, [a2 + (off + 1)])
                    ub0 = plsc.load_gather(tab, [b2 + off])
                    ub1 = plsc.load_gather(tab, [b2 + (off + 1)])
                    d0 = ua0 - ub0
                    d1 = ua1 - ub1
                    obuf[k, sl] = jnp.exp(bsq[k] - (d0 * d0 + d1 * d1))

            for k in range(TPW):
                pltpu.sync_copy(obuf.at[k], out_hbm.at[t0 + k, pl.ds(base, CP)])
            return carry

        lax.fori_loop(0, NCH, chunk_step, 0)

    return body(u, pa, pb)


def kernel(x0, v, beta, times_list, node_pairs):
    vp = jnp.pad(v.reshape(I, ND), ((0, 0), (0, NDP - ND)))
    bsq = beta[0] * beta[0]
    x0p = jnp.concatenate([
        x0.reshape(1, ND),
        jnp.full((1, 16), bsq, jnp.float32),
        jnp.zeros((1, NDP - ND - 16), jnp.float32),
    ], axis=1)

    u = pl.pallas_call(
        _tc_body,
        grid=(NDP // NB,),
        in_specs=[
            pl.BlockSpec((T, 1), lambda i: (0, 0)),
            pl.BlockSpec((I, NB), lambda i: (0, i)),
            pl.BlockSpec((1, NB), lambda i: (0, i)),
        ],
        out_specs=pl.BlockSpec((T, NB), lambda i: (0, i)),
        out_shape=jax.ShapeDtypeStruct((T, NDP), jnp.float32),
    )(times_list.reshape(T, 1), vp, x0p)

    return _sc_pair_kernel(u, node_pairs[0], node_pairs[1])


# speedup vs baseline: 1.8578x; 1.0880x over previous
import functools

import jax
import jax.numpy as jnp
from jax import lax
from jax.experimental import pallas as pl
from jax.experimental.pallas import tpu as pltpu
from jax.experimental.pallas import tpu_sc as plsc

N = 10000
D = 2
I = 32
T = 128
P = 100000
ND = N * D
NDP = 20480
NB = 5120
NC = 2
NS = 16
NW = NC * NS
TPW = T // NW
CP = 4096
NCH = 25


def _tc_body(times_ref, vp_ref, x0p_ref, u_ref):
    t = times_ref[:]
    j = lax.broadcasted_iota(jnp.int32, (T, I), 1).astype(jnp.float32)
    s = jnp.float32(1.0 / I)
    w = jnp.clip(t - j * s, 0.0, s)
    u = lax.dot_general(w, vp_ref[:], (((1,), (0,)), ((), ())),
                        precision=lax.Precision.HIGHEST,
                        preferred_element_type=jnp.float32)
    u_ref[:] = u + x0p_ref[:]


def _sc_pair_kernel(u, pa, pb):
    mesh = plsc.VectorSubcoreMesh(core_axis_name="c", subcore_axis_name="s")

    @functools.partial(
        pl.kernel,
        mesh=mesh,
        compiler_params=pltpu.CompilerParams(
            use_tc_tiling_on_sc=False, needs_layout_passes=False),
        out_type=jax.ShapeDtypeStruct((T, P), jnp.float32),
        scratch_types=[
            pltpu.VMEM((TPW * NDP,), jnp.float32),
            pltpu.VMEM((CP,), jnp.int32),
            pltpu.VMEM((CP,), jnp.int32),
            pltpu.VMEM((TPW, CP), jnp.float32),
        ],
    )
    def body(u_hbm, pa_hbm, pb_hbm, out_hbm, tab, ia, ib, obuf):
        wid = lax.axis_index("s") * NC + lax.axis_index("c")
        t0 = wid * TPW
        for k in range(TPW):
            pltpu.sync_copy(u_hbm.at[t0 + k], tab.at[pl.ds(k * NDP, NDP)])

        bsq = [tab[pl.ds(k * NDP + ND, 16)] for k in range(TPW)]

        def chunk_step(c, carry):
            base = jnp.minimum(c * CP, P - CP)
            pltpu.sync_copy(pa_hbm.at[pl.ds(base, CP)], ia)
            pltpu.sync_copy(pb_hbm.at[pl.ds(base, CP)], ib)

            @plsc.parallel_loop(0, CP // 16, unroll=4)
            def group_step(g):
                sl = pl.ds(g * 16, 16)
                a2 = ia[sl] * 2
                b2 = ib[sl] * 2
                for k in range(TPW):
                    off = k * NDP
                    ua0 = plsc.load_gather(tab, [a2 + off])
                    ua1 = plsc.load_gather(tab, [a2 + (off + 1)])
                    ub0 = plsc.load_gather(tab, [b2 + off])
                    ub1 = plsc.load_gather(tab, [b2 + (off + 1)])
                    d0 = ua0 - ub0
                    d1 = ua1 - ub1
                    obuf[k, sl] = jnp.exp(bsq[k] - (d0 * d0 + d1 * d1))

            for k in range(TPW):
                pltpu.sync_copy(obuf.at[k], out_hbm.at[t0 + k, pl.ds(base, CP)])
            return carry

        lax.fori_loop(0, NCH, chunk_step, 0)

    return body(u, pa, pb)


def kernel(x0, v, beta, times_list, node_pairs):
    vp = jnp.pad(v.reshape(I, ND), ((0, 0), (0, NDP - ND)))
    bsq = beta[0] * beta[0]
    x0p = jnp.concatenate([
        x0.reshape(1, ND),
        jnp.full((1, 16), bsq, jnp.float32),
        jnp.zeros((1, NDP - ND - 16), jnp.float32),
    ], axis=1)

    u = pl.pallas_call(
        _tc_body,
        grid=(NDP // NB,),
        in_specs=[
            pl.BlockSpec((T, 1), lambda i: (0, 0)),
            pl.BlockSpec((I, NB), lambda i: (0, i)),
            pl.BlockSpec((1, NB), lambda i: (0, i)),
        ],
        out_specs=pl.BlockSpec((T, NB), lambda i: (0, i)),
        out_shape=jax.ShapeDtypeStruct((T, NDP), jnp.float32),
    )(times_list.reshape(T, 1), vp, x0p)

    return _sc_pair_kernel(u, node_pairs[0], node_pairs[1])
